# baseline (speedup 1.0000x reference)
import functools

import jax
import jax.numpy as jnp
from jax import lax
from jax.experimental import pallas as pl
from jax.experimental.pallas import tpu as pltpu
from jax.experimental.pallas import tpu_sc as plsc

N_NODES = 10000
N_EDGES = 320000
D_FEAT = 128
D_HALF = D_FEAT // 2
N_CLASSES = 40

NC = 2
NS = 16
CHUNK = 64
G = 10
NG = 32
N_CHUNKS = G * NG
E_PER_TILE = N_CHUNKS * CHUNK
E_PAD = NS * E_PER_TILE
N_PER_TILE = 632
N_PAD = NS * N_PER_TILE


def _sc_aggregate(x_lo, x_hi, src3, dst3, z_feat, z_deg, ones_h):
  mesh = plsc.VectorSubcoreMesh(core_axis_name="c", subcore_axis_name="s")

  @functools.partial(
      pl.kernel,
      out_type=[
          jax.ShapeDtypeStruct((NC, NS, N_PER_TILE, D_HALF), jnp.float32),
          jax.ShapeDtypeStruct((NS, N_PER_TILE), jnp.float32),
      ],
      mesh=mesh,
      compiler_params=pltpu.CompilerParams(use_tc_tiling_on_sc=False),
      scratch_types=[
          pltpu.VMEM((G, CHUNK), jnp.int32),
          pltpu.VMEM((G, CHUNK), jnp.int32),
          [pltpu.VMEM((CHUNK, D_HALF), jnp.float32) for _ in range(G)],
          pltpu.VMEM((CHUNK,), jnp.float32),
          pltpu.VMEM((N_PER_TILE, D_HALF), jnp.float32),
          pltpu.VMEM((N_PER_TILE,), jnp.float32),
          pltpu.VMEM_SHARED((N_PAD, D_HALF), jnp.float32),
          pltpu.VMEM_SHARED((N_PAD,), jnp.float32),
          [pltpu.SemaphoreType.DMA for _ in range(G)],
          pltpu.SemaphoreType.DMA,
          pltpu.SemaphoreType.DMA,
      ],
  )
  def body(xlo_hbm, xhi_hbm, src_hbm, dst_hbm, zf_hbm, zd_hbm, ones_hbm,
           agg_out, deg_out,
           idx_s, idx_d, rows, ones_v, stg, stg_d, agg_sh, deg_sh,
           sem_g, sem_s, sem_o):
    c = lax.axis_index("c")
    s = lax.axis_index("s")
    base_n = s * N_PER_TILE

    pltpu.sync_copy(zf_hbm, stg)
    pltpu.sync_copy(zd_hbm, stg_d)
    pltpu.sync_copy(stg, agg_sh.at[pl.ds(base_n, N_PER_TILE)])
    pltpu.sync_copy(stg_d, deg_sh.at[pl.ds(base_n, N_PER_TILE)])
    pltpu.sync_copy(ones_hbm, ones_v)
    plsc.subcore_barrier()

    def sweep(x_half, with_deg):
      def main_body(i, carry):
        pltpu.sync_copy(src_hbm.at[s, i], idx_s)
        pltpu.sync_copy(dst_hbm.at[s, i], idx_d)
        gd = [pltpu.async_copy(x_half.at[idx_s.at[b]], rows[b],
                               sem_g[b]) for b in range(G)]
        drains = []
        for b in range(G):
          gd[b].wait()
          drains.append(pltpu.async_copy(rows[b], agg_sh.at[idx_d.at[b]],
                                         sem_s, add=True))
          if with_deg:
            drains.append(pltpu.async_copy(ones_v, deg_sh.at[idx_d.at[b]],
                                           sem_o, add=True))
        for d in drains:
          d.wait()
        return carry

      lax.fori_loop(0, NG, main_body, 0)

    @pl.when(c == 0)
    def _():
      sweep(xlo_hbm, True)

    @pl.when(c == 1)
    def _():
      sweep(xhi_hbm, False)

    plsc.subcore_barrier()

    pltpu.sync_copy(agg_sh.at[pl.ds(base_n, N_PER_TILE)], agg_out.at[c, s])

    @pl.when(c == 0)
    def _():
      pltpu.sync_copy(deg_sh.at[pl.ds(base_n, N_PER_TILE)], deg_out.at[s])

  return body(x_lo, x_hi, src3, dst3, z_feat, z_deg, ones_h)


ROWS_BLK = N_PER_TILE


def _xwr_body(x_ref, wr_ref, b_ref, out_ref):
  out_ref[...] = (jnp.dot(x_ref[...], wr_ref[...],
                          preferred_element_type=jnp.float32) + b_ref[...])


def _xwr(x, wr_t, b2d):
  blk = 1000
  return pl.pallas_call(
      _xwr_body,
      grid=(N_NODES // blk,),
      in_specs=[
          pl.BlockSpec((blk, D_FEAT), lambda i: (i, 0)),
          pl.BlockSpec((D_FEAT, N_CLASSES), lambda i: (0, 0)),
          pl.BlockSpec((1, N_CLASSES), lambda i: (0, 0)),
      ],
      out_specs=pl.BlockSpec((blk, N_CLASSES), lambda i: (i, 0)),
      out_shape=jax.ShapeDtypeStruct((N_NODES, N_CLASSES), jnp.float32),
  )(x, wr_t, b2d)


def _finalize_body(agg_ref, deg_ref, hr_ref, wl_ref, out_ref):
  a = jnp.concatenate([agg_ref[0], agg_ref[1]], axis=1)
  mean = a / jnp.maximum(deg_ref[...], 1.0)
  h = (jnp.dot(mean, wl_ref[...], preferred_element_type=jnp.float32)
       + hr_ref[...])
  m = jnp.max(h, axis=1, keepdims=True)
  lse = jnp.log(jnp.sum(jnp.exp(h - m), axis=1, keepdims=True)) + m
  out_ref[...] = h - lse


def _finalize(agg, deg, hr, wl_t):
  grid = (N_PAD // ROWS_BLK,)
  return pl.pallas_call(
      _finalize_body,
      grid=grid,
      in_specs=[
          pl.BlockSpec((NC, ROWS_BLK, D_HALF), lambda i: (0, i, 0)),
          pl.BlockSpec((ROWS_BLK, 1), lambda i: (i, 0)),
          pl.BlockSpec((ROWS_BLK, N_CLASSES), lambda i: (i, 0)),
          pl.BlockSpec((D_FEAT, N_CLASSES), lambda i: (0, 0)),
      ],
      out_specs=pl.BlockSpec((ROWS_BLK, N_CLASSES), lambda i: (i, 0)),
      out_shape=jax.ShapeDtypeStruct((N_PAD, N_CLASSES), jnp.float32),
  )(agg, deg, hr, wl_t)


def kernel(x, edge_index, W_l, W_r, b_l):
  src = edge_index[0].astype(jnp.int32)
  dst = edge_index[1].astype(jnp.int32)
  pad = E_PAD - N_EDGES
  pad_src = jnp.arange(pad, dtype=jnp.int32) % N_NODES
  pad_dst = N_NODES + (jnp.arange(pad, dtype=jnp.int32) % (N_PAD - N_NODES))
  src3 = jnp.concatenate([src, pad_src]).reshape(NS, NG, G, CHUNK)
  dst3 = jnp.concatenate([dst, pad_dst]).reshape(NS, NG, G, CHUNK)
  x_lo = x[:, :D_HALF]
  x_hi = x[:, D_HALF:]
  z_feat = jnp.zeros((N_PER_TILE, D_HALF), jnp.float32)
  z_deg = jnp.zeros((N_PER_TILE,), jnp.float32)
  ones_h = jnp.ones((CHUNK,), jnp.float32)
  hr = _xwr(x, W_r.T, b_l.reshape(1, N_CLASSES))
  agg, deg = _sc_aggregate(x_lo, x_hi, src3, dst3, z_feat, z_deg, ones_h)
  agg = agg.reshape(NC, N_PAD, D_HALF)
  deg = deg.reshape(N_PAD, 1)
  out = _finalize(agg, deg, hr, W_l.T)
  return out[:N_NODES]


# speedup vs baseline: 1.0501x; 1.0501x over previous
import functools

import jax
import jax.numpy as jnp
from jax import lax
from jax.experimental import pallas as pl
from jax.experimental.pallas import tpu as pltpu
from jax.experimental.pallas import tpu_sc as plsc

N_NODES = 10000
N_EDGES = 320000
D_FEAT = 128
D_HALF = D_FEAT // 2
N_CLASSES = 40

NC = 2
NS = 16
CHUNK = 64
G = 10
NHALF = 2
NG = 16
N_CHUNKS = G * NHALF * NG
E_PER_TILE = N_CHUNKS * CHUNK
E_PAD = NS * E_PER_TILE
N_PER_TILE = 632
N_PAD = NS * N_PER_TILE


def _sc_aggregate(x_lo, x_hi, ed, z_feat, z_deg, ones_h):
  mesh = plsc.VectorSubcoreMesh(core_axis_name="c", subcore_axis_name="s")

  @functools.partial(
      pl.kernel,
      out_type=[
          jax.ShapeDtypeStruct((NC, NS, N_PER_TILE, D_HALF), jnp.float32),
          jax.ShapeDtypeStruct((NS, N_PER_TILE), jnp.float32),
      ],
      mesh=mesh,
      compiler_params=pltpu.CompilerParams(use_tc_tiling_on_sc=False),
      scratch_types=[
          pltpu.VMEM((2, NHALF * G, CHUNK), jnp.int32),
          [pltpu.VMEM((CHUNK, D_HALF), jnp.float32) for _ in range(G)],
          pltpu.VMEM((CHUNK,), jnp.float32),
          pltpu.VMEM((N_PER_TILE, D_HALF), jnp.float32),
          pltpu.VMEM((N_PER_TILE,), jnp.float32),
          pltpu.VMEM_SHARED((N_PAD, D_HALF), jnp.float32),
          pltpu.VMEM_SHARED((N_PAD,), jnp.float32),
          [pltpu.SemaphoreType.DMA for _ in range(G)],
          pltpu.SemaphoreType.DMA,
          pltpu.SemaphoreType.DMA,
      ],
  )
  def body(xlo_hbm, xhi_hbm, ed_hbm, zf_hbm, zd_hbm, ones_hbm,
           agg_out, deg_out,
           idx2, rows, ones_v, stg, stg_d, agg_sh, deg_sh,
           sem_g, sem_s, sem_o):
    c = lax.axis_index("c")
    s = lax.axis_index("s")
    base_n = s * N_PER_TILE

    pltpu.sync_copy(zf_hbm, stg)
    pltpu.sync_copy(zd_hbm, stg_d)
    pltpu.sync_copy(stg, agg_sh.at[pl.ds(base_n, N_PER_TILE)])
    pltpu.sync_copy(stg_d, deg_sh.at[pl.ds(base_n, N_PER_TILE)])
    pltpu.sync_copy(ones_hbm, ones_v)
    plsc.subcore_barrier()

    def sweep(x_half, with_deg):
      def main_body(i, carry):
        pltpu.sync_copy(ed_hbm.at[s, i], idx2)
        for half in range(NHALF):
          gd = [pltpu.async_copy(x_half.at[idx2.at[0, half * G + b]], rows[b],
                                 sem_g[b]) for b in range(G)]
          drains = []
          for b in range(G):
            gd[b].wait()
            drains.append(pltpu.async_copy(
                rows[b], agg_sh.at[idx2.at[1, half * G + b]],
                sem_s, add=True))
            if with_deg:
              drains.append(pltpu.async_copy(
                  ones_v, deg_sh.at[idx2.at[1, half * G + b]],
                  sem_o, add=True))
          for d in drains:
            d.wait()
        return carry

      lax.fori_loop(0, NG, main_body, 0)

    @pl.when(c == 0)
    def _():
      sweep(xlo_hbm, True)

    @pl.when(c == 1)
    def _():
      sweep(xhi_hbm, False)

    plsc.subcore_barrier()

    pltpu.sync_copy(agg_sh.at[pl.ds(base_n, N_PER_TILE)], agg_out.at[c, s])

    @pl.when(c == 0)
    def _():
      pltpu.sync_copy(deg_sh.at[pl.ds(base_n, N_PER_TILE)], deg_out.at[s])

  return body(x_lo, x_hi, ed, z_feat, z_deg, ones_h)


ROWS_BLK = N_PER_TILE


def _finalize_body(agg_ref, deg_ref, x_ref, wl_ref, wr_ref, b_ref, out_ref):
  a = jnp.concatenate([agg_ref[0], agg_ref[1]], axis=1)
  mean = a / jnp.maximum(deg_ref[...], 1.0)
  h = (jnp.dot(mean, wl_ref[...], preferred_element_type=jnp.float32)
       + jnp.dot(x_ref[...], wr_ref[...], preferred_element_type=jnp.float32)
       + b_ref[...])
  m = jnp.max(h, axis=1, keepdims=True)
  lse = jnp.log(jnp.sum(jnp.exp(h - m), axis=1, keepdims=True)) + m
  out_ref[...] = h - lse


def _finalize(agg, deg, x, wl_t, wr_t, b2d):
  grid = (N_PAD // ROWS_BLK,)
  return pl.pallas_call(
      _finalize_body,
      grid=grid,
      in_specs=[
          pl.BlockSpec((NC, ROWS_BLK, D_HALF), lambda i: (0, i, 0)),
          pl.BlockSpec((ROWS_BLK, 1), lambda i: (i, 0)),
          pl.BlockSpec((ROWS_BLK, D_FEAT), lambda i: (i, 0)),
          pl.BlockSpec((D_FEAT, N_CLASSES), lambda i: (0, 0)),
          pl.BlockSpec((D_FEAT, N_CLASSES), lambda i: (0, 0)),
          pl.BlockSpec((1, N_CLASSES), lambda i: (0, 0)),
      ],
      out_specs=pl.BlockSpec((ROWS_BLK, N_CLASSES), lambda i: (i, 0)),
      out_shape=jax.ShapeDtypeStruct((N_PAD, N_CLASSES), jnp.float32),
  )(agg, deg, x, wl_t, wr_t, b2d)


def kernel(x, edge_index, W_l, W_r, b_l):
  src = edge_index[0].astype(jnp.int32)
  dst = edge_index[1].astype(jnp.int32)
  pad = E_PAD - N_EDGES
  pad_src = jnp.arange(pad, dtype=jnp.int32) % N_NODES
  pad_dst = N_NODES + (jnp.arange(pad, dtype=jnp.int32) % (N_PAD - N_NODES))
  src3 = jnp.concatenate([src, pad_src]).reshape(NS, NG, NHALF * G, CHUNK)
  dst3 = jnp.concatenate([dst, pad_dst]).reshape(NS, NG, NHALF * G, CHUNK)
  ed = jnp.stack([src3, dst3], axis=2)
  x_lo = x[:, :D_HALF]
  x_hi = x[:, D_HALF:]
  z_feat = jnp.zeros((N_PER_TILE, D_HALF), jnp.float32)
  z_deg = jnp.zeros((N_PER_TILE,), jnp.float32)
  ones_h = jnp.ones((CHUNK,), jnp.float32)
  agg, deg = _sc_aggregate(x_lo, x_hi, ed, z_feat, z_deg, ones_h)
  agg = agg.reshape(NC, N_PAD, D_HALF)
  deg = deg.reshape(N_PAD, 1)
  out = _finalize(agg, deg, x, W_l.T, W_r.T, b_l.reshape(1, N_CLASSES))
  return out[:N_NODES]


# speedup vs baseline: 1.0685x; 1.0175x over previous
import functools

import jax
import jax.numpy as jnp
from jax import lax
from jax.experimental import pallas as pl
from jax.experimental.pallas import tpu as pltpu
from jax.experimental.pallas import tpu_sc as plsc

N_NODES = 10000
N_EDGES = 320000
D_FEAT = 128
D_HALF = D_FEAT // 2
N_CLASSES = 40

NC = 2
NS = 16
CHUNK = 64
G = 5
NHALF = 4
NG = 16
N_CHUNKS = G * NHALF * NG
E_PER_TILE = N_CHUNKS * CHUNK
E_PAD = NS * E_PER_TILE
N_PER_TILE = 632
N_PAD = NS * N_PER_TILE


def _sc_aggregate(x_lo, x_hi, ed, z_feat, z_deg, ones_h):
  mesh = plsc.VectorSubcoreMesh(core_axis_name="c", subcore_axis_name="s")

  @functools.partial(
      pl.kernel,
      out_type=[
          jax.ShapeDtypeStruct((NC, NS, N_PER_TILE, D_HALF), jnp.float32),
          jax.ShapeDtypeStruct((NS, N_PER_TILE), jnp.float32),
      ],
      mesh=mesh,
      compiler_params=pltpu.CompilerParams(use_tc_tiling_on_sc=False),
      scratch_types=[
          pltpu.VMEM((2, NHALF * G, CHUNK), jnp.int32),
          [pltpu.VMEM((CHUNK, D_HALF), jnp.float32) for _ in range(2 * G)],
          pltpu.VMEM((CHUNK,), jnp.float32),
          pltpu.VMEM((N_PER_TILE, D_HALF), jnp.float32),
          pltpu.VMEM((N_PER_TILE,), jnp.float32),
          pltpu.VMEM_SHARED((N_PAD, D_HALF), jnp.float32),
          pltpu.VMEM_SHARED((N_PAD,), jnp.float32),
          [pltpu.SemaphoreType.DMA for _ in range(2 * G)],
          pltpu.SemaphoreType.DMA,
          pltpu.SemaphoreType.DMA,
      ],
  )
  def body(xlo_hbm, xhi_hbm, ed_hbm, zf_hbm, zd_hbm, ones_hbm,
           agg_out, deg_out,
           idx2, rows, ones_v, stg, stg_d, agg_sh, deg_sh,
           sem_g, sem_s, sem_o):
    c = lax.axis_index("c")
    s = lax.axis_index("s")
    base_n = s * N_PER_TILE

    pltpu.sync_copy(zf_hbm, stg)
    pltpu.sync_copy(zd_hbm, stg_d)
    pltpu.sync_copy(stg, agg_sh.at[pl.ds(base_n, N_PER_TILE)])
    pltpu.sync_copy(stg_d, deg_sh.at[pl.ds(base_n, N_PER_TILE)])
    pltpu.sync_copy(ones_hbm, ones_v)
    plsc.subcore_barrier()

    def sweep(x_half, with_deg):
      def main_body(i, carry):
        pltpu.sync_copy(ed_hbm.at[s, i], idx2)
        prev = []
        for half in range(NHALF):
          base = half * G
          off = (half % 2) * G
          gd = [pltpu.async_copy(x_half.at[idx2.at[0, base + b]],
                                 rows[off + b], sem_g[off + b])
                for b in range(G)]
          for d in prev:
            d.wait()
          prev = []
          for b in range(G):
            gd[b].wait()
            prev.append(pltpu.async_copy(
                rows[off + b], agg_sh.at[idx2.at[1, base + b]],
                sem_s, add=True))
            if with_deg:
              prev.append(pltpu.async_copy(
                  ones_v, deg_sh.at[idx2.at[1, base + b]],
                  sem_o, add=True))
        for d in prev:
          d.wait()
        return carry

      lax.fori_loop(0, NG, main_body, 0)

    @pl.when(c == 0)
    def _():
      sweep(xlo_hbm, True)

    @pl.when(c == 1)
    def _():
      sweep(xhi_hbm, False)

    plsc.subcore_barrier()

    pltpu.sync_copy(agg_sh.at[pl.ds(base_n, N_PER_TILE)], agg_out.at[c, s])

    @pl.when(c == 0)
    def _():
      pltpu.sync_copy(deg_sh.at[pl.ds(base_n, N_PER_TILE)], deg_out.at[s])

  return body(x_lo, x_hi, ed, z_feat, z_deg, ones_h)


ROWS_BLK = N_PER_TILE


def _finalize_body(agg_ref, deg_ref, x_ref, wl_ref, wr_ref, b_ref, out_ref):
  a = jnp.concatenate([agg_ref[0], agg_ref[1]], axis=1)
  mean = a / jnp.maximum(deg_ref[...], 1.0)
  h = (jnp.dot(mean, wl_ref[...], preferred_element_type=jnp.float32)
       + jnp.dot(x_ref[...], wr_ref[...], preferred_element_type=jnp.float32)
       + b_ref[...])
  m = jnp.max(h, axis=1, keepdims=True)
  lse = jnp.log(jnp.sum(jnp.exp(h - m), axis=1, keepdims=True)) + m
  out_ref[...] = h - lse


def _finalize(agg, deg, x, wl_t, wr_t, b2d):
  grid = (N_PAD // ROWS_BLK,)
  return pl.pallas_call(
      _finalize_body,
      grid=grid,
      in_specs=[
          pl.BlockSpec((NC, ROWS_BLK, D_HALF), lambda i: (0, i, 0)),
          pl.BlockSpec((ROWS_BLK, 1), lambda i: (i, 0)),
          pl.BlockSpec((ROWS_BLK, D_FEAT), lambda i: (i, 0)),
          pl.BlockSpec((D_FEAT, N_CLASSES), lambda i: (0, 0)),
          pl.BlockSpec((D_FEAT, N_CLASSES), lambda i: (0, 0)),
          pl.BlockSpec((1, N_CLASSES), lambda i: (0, 0)),
      ],
      out_specs=pl.BlockSpec((ROWS_BLK, N_CLASSES), lambda i: (i, 0)),
      out_shape=jax.ShapeDtypeStruct((N_PAD, N_CLASSES), jnp.float32),
  )(agg, deg, x, wl_t, wr_t, b2d)


def kernel(x, edge_index, W_l, W_r, b_l):
  src = edge_index[0].astype(jnp.int32)
  dst = edge_index[1].astype(jnp.int32)
  pad = E_PAD - N_EDGES
  pad_src = jnp.arange(pad, dtype=jnp.int32) % N_NODES
  pad_dst = N_NODES + (jnp.arange(pad, dtype=jnp.int32) % (N_PAD - N_NODES))
  src3 = jnp.concatenate([src, pad_src]).reshape(NS, NG, NHALF * G, CHUNK)
  dst3 = jnp.concatenate([dst, pad_dst]).reshape(NS, NG, NHALF * G, CHUNK)
  ed = jnp.stack([src3, dst3], axis=2)
  x_lo = x[:, :D_HALF]
  x_hi = x[:, D_HALF:]
  z_feat = jnp.zeros((N_PER_TILE, D_HALF), jnp.float32)
  z_deg = jnp.zeros((N_PER_TILE,), jnp.float32)
  ones_h = jnp.ones((CHUNK,), jnp.float32)
  agg, deg = _sc_aggregate(x_lo, x_hi, ed, z_feat, z_deg, ones_h)
  agg = agg.reshape(NC, N_PAD, D_HALF)
  deg = deg.reshape(N_PAD, 1)
  out = _finalize(agg, deg, x, W_l.T, W_r.T, b_l.reshape(1, N_CLASSES))
  return out[:N_NODES]


# speedup vs baseline: 1.0981x; 1.0277x over previous
import functools

import jax
import jax.numpy as jnp
from jax import lax
from jax.experimental import pallas as pl
from jax.experimental.pallas import tpu as pltpu
from jax.experimental.pallas import tpu_sc as plsc

N_NODES = 10000
N_EDGES = 320000
D_FEAT = 128
D_HALF = D_FEAT // 2
N_CLASSES = 40

NC = 2
NS = 16
CHUNK = 64
G = 5
NHALF = 8
NG = 8
N_CHUNKS = G * NHALF * NG
E_PER_TILE = N_CHUNKS * CHUNK
E_PAD = NS * E_PER_TILE
N_PER_TILE = 632
N_PAD = NS * N_PER_TILE


def _sc_aggregate(x_lo, x_hi, ed, z_feat, z_deg, ones_h):
  mesh = plsc.VectorSubcoreMesh(core_axis_name="c", subcore_axis_name="s")

  @functools.partial(
      pl.kernel,
      out_type=[
          jax.ShapeDtypeStruct((NC, NS, N_PER_TILE, D_HALF), jnp.float32),
          jax.ShapeDtypeStruct((NS, N_PER_TILE), jnp.float32),
      ],
      mesh=mesh,
      compiler_params=pltpu.CompilerParams(use_tc_tiling_on_sc=False),
      scratch_types=[
          pltpu.VMEM((2, NHALF * G, CHUNK), jnp.int32),
          [pltpu.VMEM((CHUNK, D_HALF), jnp.float32) for _ in range(2 * G)],
          pltpu.VMEM((CHUNK,), jnp.float32),
          pltpu.VMEM((N_PER_TILE, D_HALF), jnp.float32),
          pltpu.VMEM((N_PER_TILE,), jnp.float32),
          pltpu.VMEM_SHARED((N_PAD, D_HALF), jnp.float32),
          pltpu.VMEM_SHARED((N_PAD,), jnp.float32),
          [pltpu.SemaphoreType.DMA for _ in range(2 * G)],
          pltpu.SemaphoreType.DMA,
          pltpu.SemaphoreType.DMA,
      ],
  )
  def body(xlo_hbm, xhi_hbm, ed_hbm, zf_hbm, zd_hbm, ones_hbm,
           agg_out, deg_out,
           idx2, rows, ones_v, stg, stg_d, agg_sh, deg_sh,
           sem_g, sem_s, sem_o):
    c = lax.axis_index("c")
    s = lax.axis_index("s")
    base_n = s * N_PER_TILE

    pltpu.sync_copy(zf_hbm, stg)
    pltpu.sync_copy(zd_hbm, stg_d)
    pltpu.sync_copy(stg, agg_sh.at[pl.ds(base_n, N_PER_TILE)])
    pltpu.sync_copy(stg_d, deg_sh.at[pl.ds(base_n, N_PER_TILE)])
    pltpu.sync_copy(ones_hbm, ones_v)
    plsc.subcore_barrier()

    def sweep(x_half, with_deg):
      def main_body(i, carry):
        pltpu.sync_copy(ed_hbm.at[s, i], idx2)
        prev = []
        for half in range(NHALF):
          base = half * G
          off = (half % 2) * G
          gd = [pltpu.async_copy(x_half.at[idx2.at[0, base + b]],
                                 rows[off + b], sem_g[off + b])
                for b in range(G)]
          for d in prev:
            d.wait()
          prev = []
          for b in range(G):
            gd[b].wait()
            prev.append(pltpu.async_copy(
                rows[off + b], agg_sh.at[idx2.at[1, base + b]],
                sem_s, add=True))
            if with_deg:
              prev.append(pltpu.async_copy(
                  ones_v, deg_sh.at[idx2.at[1, base + b]],
                  sem_o, add=True))
        for d in prev:
          d.wait()
        return carry

      lax.fori_loop(0, NG, main_body, 0)

    @pl.when(c == 0)
    def _():
      sweep(xlo_hbm, True)

    @pl.when(c == 1)
    def _():
      sweep(xhi_hbm, False)

    plsc.subcore_barrier()

    pltpu.sync_copy(agg_sh.at[pl.ds(base_n, N_PER_TILE)], agg_out.at[c, s])

    @pl.when(c == 0)
    def _():
      pltpu.sync_copy(deg_sh.at[pl.ds(base_n, N_PER_TILE)], deg_out.at[s])

  return body(x_lo, x_hi, ed, z_feat, z_deg, ones_h)


ROWS_BLK = N_PER_TILE


def _finalize_body(agg_ref, deg_ref, x_ref, wl_ref, wr_ref, b_ref, out_ref):
  a = jnp.concatenate([agg_ref[0], agg_ref[1]], axis=1)
  mean = a / jnp.maximum(deg_ref[...], 1.0)
  h = (jnp.dot(mean, wl_ref[...], preferred_element_type=jnp.float32)
       + jnp.dot(x_ref[...], wr_ref[...], preferred_element_type=jnp.float32)
       + b_ref[...])
  m = jnp.max(h, axis=1, keepdims=True)
  lse = jnp.log(jnp.sum(jnp.exp(h - m), axis=1, keepdims=True)) + m
  out_ref[...] = h - lse


def _finalize(agg, deg, x, wl_t, wr_t, b2d):
  grid = (N_PAD // ROWS_BLK,)
  return pl.pallas_call(
      _finalize_body,
      grid=grid,
      in_specs=[
          pl.BlockSpec((NC, ROWS_BLK, D_HALF), lambda i: (0, i, 0)),
          pl.BlockSpec((ROWS_BLK, 1), lambda i: (i, 0)),
          pl.BlockSpec((ROWS_BLK, D_FEAT), lambda i: (i, 0)),
          pl.BlockSpec((D_FEAT, N_CLASSES), lambda i: (0, 0)),
          pl.BlockSpec((D_FEAT, N_CLASSES), lambda i: (0, 0)),
          pl.BlockSpec((1, N_CLASSES), lambda i: (0, 0)),
      ],
      out_specs=pl.BlockSpec((ROWS_BLK, N_CLASSES), lambda i: (i, 0)),
      out_shape=jax.ShapeDtypeStruct((N_NODES, N_CLASSES), jnp.float32),
  )(agg, deg, x, wl_t, wr_t, b2d)


def kernel(x, edge_index, W_l, W_r, b_l):
  src = edge_index[0].astype(jnp.int32)
  dst = edge_index[1].astype(jnp.int32)
  pad = E_PAD - N_EDGES
  pad_src = jnp.arange(pad, dtype=jnp.int32) % N_NODES
  pad_dst = N_NODES + (jnp.arange(pad, dtype=jnp.int32) % (N_PAD - N_NODES))
  src3 = jnp.concatenate([src, pad_src]).reshape(NS, NG, NHALF * G, CHUNK)
  dst3 = jnp.concatenate([dst, pad_dst]).reshape(NS, NG, NHALF * G, CHUNK)
  ed = jnp.stack([src3, dst3], axis=2)
  x_lo = x[:, :D_HALF]
  x_hi = x[:, D_HALF:]
  z_feat = jnp.zeros((N_PER_TILE, D_HALF), jnp.float32)
  z_deg = jnp.zeros((N_PER_TILE,), jnp.float32)
  ones_h = jnp.ones((CHUNK,), jnp.float32)
  agg, deg = _sc_aggregate(x_lo, x_hi, ed, z_feat, z_deg, ones_h)
  agg = agg.reshape(NC, N_PAD, D_HALF)
  deg = deg.reshape(N_PAD, 1)
  return _finalize(agg, deg, x, W_l.T, W_r.T, b_l.reshape(1, N_CLASSES))
